# baseline (speedup 1.0000x reference)
import functools

import jax
import jax.numpy as jnp
from jax import lax
from jax.experimental import pallas as pl
from jax.experimental.pallas import tpu as pltpu
from jax.experimental.pallas import tpu_sc as plsc

N = 50000
IN_DIM = 1024
D = 512
H = 8
HD = D // H
NT = 6
KK = N // 10
TILE = 2000
NTILES = N // TILE
CHUNK = 1024
NPAD = 5120
NCHUNK = NPAD // CHUNK
NWORK = 32
GROWS = NT * NPAD // NWORK
GCH = 120
NREG = N // 16
_SC_MESH = lambda: plsc.VectorSubcoreMesh(core_axis_name="c",
                                          subcore_axis_name="s")


def _blockdiag(dtype=jnp.float32):
    r = jax.lax.broadcasted_iota(jnp.int32, (H, D), 0)
    c = jax.lax.broadcasted_iota(jnp.int32, (H, D), 1)
    return (c // HD == r).astype(dtype)


def _ln_rows(o, g, b):
    mu = jnp.mean(o, axis=1, keepdims=True)
    va = jnp.mean((o - mu) ** 2, axis=1, keepdims=True)
    return (o - mu) / jnp.sqrt(va + 1e-6) * g + b


def _prep_body(protos_ref, Wp_ref, bp_ref, Wq_ref, bq_ref, pp_ref, qsel_ref):
    pp = protos_ref[0] @ Wp_ref[...] + bp_ref[...]
    qsel = pp @ Wq_ref[...] + bq_ref[...]
    pp_ref[...] = pp
    qsel_ref[...] = qsel


def _prep(protos, Wp, bp, Wq, bq):
    return pl.pallas_call(
        _prep_body,
        out_shape=[jax.ShapeDtypeStruct((NT, D), jnp.float32)] * 2,
    )(protos, Wp, bp, Wq, bq)


def _feat_body(x_ref, Wfe_ref, bfe_ref, g_ref, b_ref, Wk_ref, bk_ref,
               qsel_ref, feat_ref, sc_ref, fs_ref):
    i = pl.program_id(0)
    h = jnp.maximum(x_ref[...] @ Wfe_ref[...] + bfe_ref[...], 0.0)
    f = _ln_rows(h, g_ref[...], b_ref[...])
    feat_ref[...] = f
    kp = f @ Wk_ref[...] + bk_ref[...]
    sc_ref[0] = jax.lax.dot_general(
        qsel_ref[...], kp, (((1,), (1,)), ((), ()))) * (1.0 / 64.0)

    @pl.when(i == 0)
    def _():
        fs_ref[...] = jnp.zeros_like(fs_ref)

    fs_ref[...] += jnp.sum(f, axis=0, keepdims=True)


def _feat(x, Wfe, bfe, g, b, Wk, bk, qsel):
    return pl.pallas_call(
        _feat_body,
        grid=(NTILES,),
        in_specs=[
            pl.BlockSpec((TILE, IN_DIM), lambda i: (i, 0)),
            pl.BlockSpec((IN_DIM, D), lambda i: (0, 0)),
            pl.BlockSpec((1, D), lambda i: (0, 0)),
            pl.BlockSpec((1, D), lambda i: (0, 0)),
            pl.BlockSpec((1, D), lambda i: (0, 0)),
            pl.BlockSpec((D, D), lambda i: (0, 0)),
            pl.BlockSpec((1, D), lambda i: (0, 0)),
            pl.BlockSpec((NT, D), lambda i: (0, 0)),
        ],
        out_specs=[
            pl.BlockSpec((TILE, D), lambda i: (i, 0)),
            pl.BlockSpec((1, NT, TILE), lambda i: (i, 0, 0)),
            pl.BlockSpec((1, D), lambda i: (0, 0)),
        ],
        out_shape=[
            jax.ShapeDtypeStruct((N, D), jnp.float32),
            jax.ShapeDtypeStruct((NTILES, NT, TILE), jnp.float32),
            jax.ShapeDtypeStruct((1, D), jnp.float32),
        ],
    )(x, Wfe, bfe, g, b, Wk, bk, qsel)


def _thr_body(sc_ref, thr_ref, need_ref):
    s = sc_ref[...]
    u = jax.lax.bitcast_convert_type(s, jnp.int32)
    v = u ^ ((u >> 31) & jnp.int32(0x7FFFFFFF))
    cnt0 = jnp.sum((v >= 0).astype(jnp.int32), axis=(0, 2), keepdims=True)
    t = jnp.where(cnt0 >= KK, jnp.int32(0), jnp.int32(-2147483648))
    t = jnp.broadcast_to(t, (1, NT, 1))
    for b in range(30, -1, -1):
        cand = t | jnp.int32(1 << b)
        cnt = jnp.sum((v >= cand).astype(jnp.int32), axis=(0, 2),
                      keepdims=True)
        t = jnp.where(cnt >= KK, cand, t)
    cgt = jnp.sum((v > t).astype(jnp.int32), axis=(0, 2), keepdims=True)
    thr_ref[...] = jnp.broadcast_to(t.reshape(NT, 1), (NT, 16))
    need_ref[...] = jnp.broadcast_to((KK - cgt).reshape(NT, 1), (NT, 16))


def _thr(scores):
    return pl.pallas_call(
        _thr_body,
        out_shape=[
            jax.ShapeDtypeStruct((NT, 16), jnp.int32),
            jax.ShapeDtypeStruct((NT, 16), jnp.int32),
        ],
    )(scores)


def _compact_body(sc_hbm, thr_hbm, need_hbm, idx_hbm,
                  buf, obuf, thr_v, need_v):
    wid = lax.axis_index("s") * 2 + lax.axis_index("c")

    @pl.when(wid < NT)
    def _():
        t = wid
        pltpu.sync_copy(thr_hbm.at[pl.ds(t * 16, 16)], thr_v)
        pltpu.sync_copy(need_hbm.at[pl.ds(t * 16, 16)], need_v)
        pltpu.sync_copy(sc_hbm.at[pl.ds(t * N, N)], buf)

        def zero_body(j, carry):
            obuf[pl.ds(j * 16, 16)] = jnp.zeros((16,), jnp.int32)
            return carry

        lax.fori_loop(0, NPAD // 16, zero_body, 0)
        tvec = thr_v[...]
        nvec = need_v[...]

        def body(i, carry):
            off, eq_off = carry
            u = buf[pl.ds(i * 16, 16)]
            v = u ^ ((u >> 31) & jnp.int32(0x7FFFFFFF))
            gt = v > tvec
            eq = v == tvec
            eqi = eq.astype(jnp.int32)
            pre = plsc.cumsum(eqi)
            eqo = jax.lax.broadcast(eq_off, (16,))
            sel_eq = eq & ((pre + eqo) <= nvec)
            m = gt | sel_eq
            mi = m.astype(jnp.int32)
            cm = plsc.cumsum(mi)
            idxv = jax.lax.broadcast(i * 16, (16,)) + lax.iota(jnp.int32, 16)
            pos = jnp.where(
                m,
                jax.lax.broadcast(off, (16,)) + cm - 1,
                jax.lax.broadcast(NPAD, (16,)) + lax.iota(jnp.int32, 16))
            plsc.store_scatter(obuf, [pos], idxv)
            return (off + jnp.sum(mi), eq_off + jnp.sum(eqi))

        lax.fori_loop(0, NREG, body, (jnp.int32(0), jnp.int32(0)))
        pltpu.sync_copy(obuf.at[pl.ds(0, NPAD)],
                        idx_hbm.at[pl.ds(t * NPAD, NPAD)])


def _compact(scores_flat, thr_flat, need_flat):
    return pl.kernel(
        _compact_body,
        out_type=jax.ShapeDtypeStruct((NT * NPAD,), jnp.int32),
        mesh=_SC_MESH(),
        compiler_params=pltpu.CompilerParams(needs_layout_passes=False),
        scratch_types=[
            pltpu.VMEM((N,), jnp.int32),
            pltpu.VMEM((NPAD + 16,), jnp.int32),
            pltpu.VMEM((16,), jnp.int32),
            pltpu.VMEM((16,), jnp.int32),
        ],
    )(scores_flat, thr_flat, need_flat)


def _gather_body(feat_hbm, idx_hbm, out_hbm, idx_v, rows0, rows1, sem0, sem1):
    wid = lax.axis_index("s") * 2 + lax.axis_index("c")
    base = wid * GROWS
    pltpu.sync_copy(idx_hbm.at[pl.ds(base, GROWS)], idx_v)
    nch = GROWS // GCH
    bufs = (rows0, rows1)
    sems = (sem0, sem1)
    pend = [None, None]
    pend[0] = pltpu.async_copy(
        feat_hbm.at[idx_v.at[pl.ds(0, GCH)]], rows0, sem0)
    for j in range(nch):
        if j + 1 < nch:
            pend[(j + 1) % 2] = pltpu.async_copy(
                feat_hbm.at[idx_v.at[pl.ds((j + 1) * GCH, GCH)]],
                bufs[(j + 1) % 2], sems[(j + 1) % 2])
        pend[j % 2].wait()
        pltpu.sync_copy(bufs[j % 2], out_hbm.at[pl.ds(base + j * GCH, GCH)])


def _gather(feat, idxf):
    return pl.kernel(
        _gather_body,
        out_type=jax.ShapeDtypeStruct((NT * NPAD, D), jnp.float32),
        mesh=_SC_MESH(),
        compiler_params=pltpu.CompilerParams(needs_layout_passes=False),
        scratch_types=[
            pltpu.VMEM((GROWS,), jnp.int32),
            pltpu.VMEM((GCH, D), jnp.float32),
            pltpu.VMEM((GCH, D), jnp.float32),
            pltpu.SemaphoreType.DMA,
            pltpu.SemaphoreType.DMA,
        ],
    )(feat, idxf)


def _tok_body(g3_ref, pp_ref, Wq_ref, bq_ref, Wk_ref, bk_ref, Wv_ref, bv_ref,
              Wo_ref, bo_ref, g_ref, bn_ref, Wc_ref, bc_ref,
              tf_ref, tl_ref, lst, vst):
    c = pl.program_id(1)

    @pl.when(c < NCHUNK)
    def _():
        sel = g3_ref[0]
        q = pp_ref[0] @ Wq_ref[0] + bq_ref[0]
        kk_ = sel @ Wk_ref[0] + bk_ref[0]
        vv = sel @ Wv_ref[0] + bv_ref[0]
        l8 = jax.lax.dot_general(kk_ * q, _blockdiag(),
                                 (((1,), (1,)), ((), ()))) * (1.0 / 8.0)
        row = c * CHUNK + jax.lax.broadcasted_iota(jnp.int32, (CHUNK, H), 0)
        l8 = jnp.where(row < KK, l8, -1e30)
        lst[pl.ds(c * CHUNK, CHUNK), :] = l8
        vst[pl.ds(c * CHUNK, CHUNK), :] = vv

    @pl.when(c == NCHUNK)
    def _():
        l8 = lst[...]
        m = jnp.max(l8, axis=0, keepdims=True)
        e = jnp.exp(l8 - m)
        z = jnp.sum(e, axis=0, keepdims=True)
        p = e / z
        pexp = p @ _blockdiag()
        comb = jnp.sum(pexp * vst[...], axis=0, keepdims=True)
        o = comb @ Wo_ref[0] + bo_ref[0]
        of = _ln_rows(o, g_ref[0], bn_ref[0])
        tf_ref[0] = of
        tl_ref[0] = of @ Wc_ref[0] + bc_ref[0]


def _tok(g3, pp, tWq, tbq, tWk, tbk, tWv, tbv, tWo, tbo, tg, tbn, Wc, bc):
    per_tok_mat = pl.BlockSpec((1, D, D), lambda t, c: (t, 0, 0))
    per_tok_vec = pl.BlockSpec((1, 1, D), lambda t, c: (t, 0, 0))
    return pl.pallas_call(
        _tok_body,
        grid=(NT, NCHUNK + 1),
        in_specs=[
            pl.BlockSpec((1, CHUNK, D),
                         lambda t, c: (t, jnp.minimum(c, NCHUNK - 1), 0)),
            per_tok_vec,
            per_tok_mat, per_tok_vec,
            per_tok_mat, per_tok_vec,
            per_tok_mat, per_tok_vec,
            per_tok_mat, per_tok_vec,
            per_tok_vec, per_tok_vec,
            pl.BlockSpec((1, D, 128), lambda t, c: (t, 0, 0)),
            pl.BlockSpec((1, 1, 128), lambda t, c: (t, 0, 0)),
        ],
        out_specs=[
            pl.BlockSpec((1, 1, D), lambda t, c: (t, 0, 0)),
            pl.BlockSpec((1, 1, 128), lambda t, c: (t, 0, 0)),
        ],
        out_shape=[
            jax.ShapeDtypeStruct((NT, 1, D), jnp.float32),
            jax.ShapeDtypeStruct((NT, 1, 128), jnp.float32),
        ],
        scratch_shapes=[
            pltpu.VMEM((NPAD, H), jnp.float32),
            pltpu.VMEM((NPAD, D), jnp.float32),
        ],
    )(g3, pp, tWq, tbq, tWk, tbk, tWv, tbv, tWo, tbo, tg, tbn, Wc, bc)


def _agg_body(tf_ref, qsel_ref, fs_ref, Wk_ref, bk_ref, Wv_ref, bv_ref,
              Wo_ref, bo_ref, g_ref, bn_ref, Ws_ref, bs_ref, out_ref):
    E_ = _blockdiag()
    ksum = fs_ref[...] @ Wk_ref[...] + N * bk_ref[...]
    tmp = qsel_ref[...] * ksum
    msel = jax.lax.dot_general(tmp, E_, (((1,), (1,)), ((), ()))) * (
        1.0 / (HD ** 0.5) / N)
    mmax = jnp.max(msel, axis=0, keepdims=True)
    e = jnp.exp(msel - mmax)
    w = e / jnp.sum(e, axis=0, keepdims=True)
    vp = tf_ref[...] @ Wv_ref[...] + bv_ref[...]
    wexp = w @ E_
    comb = jnp.sum(wexp * vp, axis=0, keepdims=True)
    o = comb @ Wo_ref[...] + bo_ref[...]
    of = _ln_rows(o, g_ref[...], bn_ref[...])
    out_ref[...] = of @ Ws_ref[...] + bs_ref[...]


def _agg(tf, qsel, fs, Wk, bk, Wv, bv, Wo, bo, g, bn, Ws, bs):
    return pl.pallas_call(
        _agg_body,
        out_shape=jax.ShapeDtypeStruct((1, 128), jnp.float32),
    )(tf, qsel, fs, Wk, bk, Wv, bv, Wo, bo, g, bn, Ws, bs)


def kernel(x, W_fe, b_fe, g_norm, bn_norm, W_proj, b_proj, prototypes,
           sel_Wq, sel_bq, sel_Wk, sel_bk, sel_Wv, sel_bv, sel_Wo, sel_bo,
           sel_g, sel_bn,
           tok_Wq, tok_bq, tok_Wk, tok_bk, tok_Wv, tok_bv, tok_Wo, tok_bo,
           tok_g, tok_bn,
           agg_Wv, agg_bv, agg_Wo, agg_bo, agg_g, agg_bn,
           tok_cls_W, tok_cls_b, slide_W, slide_b):
    r2 = lambda a: a.reshape(1, -1)

    pp, qsel = _prep(prototypes, W_proj, r2(b_proj), sel_Wq, r2(sel_bq))
    feat, scores3, fs = _feat(x, W_fe, r2(b_fe), r2(g_norm), r2(bn_norm),
                              sel_Wk, r2(sel_bk), qsel)
    thr, need = _thr(scores3)
    scores_i = jax.lax.bitcast_convert_type(scores3, jnp.int32)
    scf = scores_i.transpose(1, 0, 2).reshape(NT * N)
    idx = _compact(scf, thr.reshape(-1), need.reshape(-1))
    g3 = _gather(feat, idx).reshape(NT, NPAD, D)

    Wc = jnp.pad(tok_cls_W, ((0, 0), (0, 0), (0, 126)))
    bc = jnp.pad(tok_cls_b, ((0, 0), (0, 126))).reshape(NT, 1, 128)
    r3 = lambda a: a.reshape(NT, 1, D)
    tf, tl = _tok(g3, pp.reshape(NT, 1, D), tok_Wq, r3(tok_bq), tok_Wk,
                  r3(tok_bk), tok_Wv, r3(tok_bv), tok_Wo, r3(tok_bo),
                  r3(tok_g), r3(tok_bn), Wc, bc)

    Ws = jnp.pad(slide_W, ((0, 0), (0, 126)))
    bs = jnp.pad(r2(slide_b), ((0, 0), (0, 126)))
    sl = _agg(tf.reshape(NT, D), qsel, fs, sel_Wk, r2(sel_bk), agg_Wv,
              r2(agg_bv), agg_Wo, r2(agg_bo), r2(agg_g), r2(agg_bn), Ws, bs)

    return sl[:, :2], tl.reshape(NT, 128)[:, :2].reshape(NT, 1, 2)


# speedup vs baseline: 1.1369x; 1.1369x over previous
import functools

import jax
import jax.numpy as jnp
from jax import lax
from jax.experimental import pallas as pl
from jax.experimental.pallas import tpu as pltpu
from jax.experimental.pallas import tpu_sc as plsc

N = 50000
IN_DIM = 1024
D = 512
H = 8
HD = D // H
NT = 6
KK = N // 10
TILE = 2000
NTILES = N // TILE
CHUNK = 1024
NPAD = 5120
NCHUNK = NPAD // CHUNK
NWORK = 32
GROWS = NT * NPAD // NWORK
GCH = 120
NREG = N // 16
_SC_MESH = lambda: plsc.VectorSubcoreMesh(core_axis_name="c",
                                          subcore_axis_name="s")


def _blockdiag(dtype=jnp.float32):
    r = jax.lax.broadcasted_iota(jnp.int32, (H, D), 0)
    c = jax.lax.broadcasted_iota(jnp.int32, (H, D), 1)
    return (c // HD == r).astype(dtype)


def _ln_rows(o, g, b):
    mu = jnp.mean(o, axis=1, keepdims=True)
    va = jnp.mean((o - mu) ** 2, axis=1, keepdims=True)
    return (o - mu) / jnp.sqrt(va + 1e-6) * g + b


def _prep_body(protos_ref, Wp_ref, bp_ref, Wq_ref, bq_ref, pp_ref, qsel_ref):
    pp = protos_ref[0] @ Wp_ref[...] + bp_ref[...]
    qsel = pp @ Wq_ref[...] + bq_ref[...]
    pp_ref[...] = pp
    qsel_ref[...] = qsel


def _prep(protos, Wp, bp, Wq, bq):
    return pl.pallas_call(
        _prep_body,
        out_shape=[jax.ShapeDtypeStruct((NT, D), jnp.float32)] * 2,
    )(protos, Wp, bp, Wq, bq)


def _feat_body(x_ref, Wfe_ref, bfe_ref, g_ref, b_ref, Wk_ref, bk_ref,
               qsel_ref, feat_ref, sc_ref, fs_ref):
    i = pl.program_id(0)
    h = jnp.maximum(x_ref[...] @ Wfe_ref[...] + bfe_ref[...], 0.0)
    f = _ln_rows(h, g_ref[...], b_ref[...])
    feat_ref[...] = f
    kp = f @ Wk_ref[...] + bk_ref[...]
    sc_ref[0] = jax.lax.dot_general(
        qsel_ref[...], kp, (((1,), (1,)), ((), ()))) * (1.0 / 64.0)

    @pl.when(i == 0)
    def _():
        fs_ref[...] = jnp.zeros_like(fs_ref)

    fs_ref[...] += jnp.sum(f, axis=0, keepdims=True)


def _feat(x, Wfe, bfe, g, b, Wk, bk, qsel):
    return pl.pallas_call(
        _feat_body,
        grid=(NTILES,),
        in_specs=[
            pl.BlockSpec((TILE, IN_DIM), lambda i: (i, 0)),
            pl.BlockSpec((IN_DIM, D), lambda i: (0, 0)),
            pl.BlockSpec((1, D), lambda i: (0, 0)),
            pl.BlockSpec((1, D), lambda i: (0, 0)),
            pl.BlockSpec((1, D), lambda i: (0, 0)),
            pl.BlockSpec((D, D), lambda i: (0, 0)),
            pl.BlockSpec((1, D), lambda i: (0, 0)),
            pl.BlockSpec((NT, D), lambda i: (0, 0)),
        ],
        out_specs=[
            pl.BlockSpec((TILE, D), lambda i: (i, 0)),
            pl.BlockSpec((1, NT, TILE), lambda i: (i, 0, 0)),
            pl.BlockSpec((1, D), lambda i: (0, 0)),
        ],
        out_shape=[
            jax.ShapeDtypeStruct((N, D), jnp.float32),
            jax.ShapeDtypeStruct((NTILES, NT, TILE), jnp.float32),
            jax.ShapeDtypeStruct((1, D), jnp.float32),
        ],
    )(x, Wfe, bfe, g, b, Wk, bk, qsel)


def _thr_body(sc_ref, thr_ref, need_ref):
    s = sc_ref[...]
    u = jax.lax.bitcast_convert_type(s, jnp.int32)
    v = u ^ ((u >> 31) & jnp.int32(0x7FFFFFFF))
    cnt0 = jnp.sum((v >= 0).astype(jnp.int32), axis=(0, 2), keepdims=True)
    t = jnp.where(cnt0 >= KK, jnp.int32(0), jnp.int32(-2147483648))
    t = jnp.broadcast_to(t, (1, NT, 1))
    for b in range(30, -1, -1):
        cand = t | jnp.int32(1 << b)
        cnt = jnp.sum((v >= cand).astype(jnp.int32), axis=(0, 2),
                      keepdims=True)
        t = jnp.where(cnt >= KK, cand, t)
    cgt = jnp.sum((v > t).astype(jnp.int32), axis=(0, 2), keepdims=True)
    thr_ref[...] = jnp.broadcast_to(t.reshape(NT, 1), (NT, 16))
    need_ref[...] = jnp.broadcast_to((KK - cgt).reshape(NT, 1), (NT, 16))


def _thr(scores):
    return pl.pallas_call(
        _thr_body,
        out_shape=[
            jax.ShapeDtypeStruct((NT, 16), jnp.int32),
            jax.ShapeDtypeStruct((NT, 16), jnp.int32),
        ],
    )(scores)


def _compact_body(sc_hbm, thr_hbm, need_hbm, idx_hbm,
                  buf, obuf, thr_v, need_v):
    wid = lax.axis_index("s") * 2 + lax.axis_index("c")

    @pl.when(wid < NT)
    def _():
        t = wid
        pltpu.sync_copy(thr_hbm.at[pl.ds(t * 16, 16)], thr_v)
        pltpu.sync_copy(need_hbm.at[pl.ds(t * 16, 16)], need_v)
        pltpu.sync_copy(sc_hbm.at[pl.ds(t * N, N)], buf)

        def zero_body(j, carry):
            obuf[pl.ds(j * 16, 16)] = jnp.zeros((16,), jnp.int32)
            return carry

        lax.fori_loop(0, NPAD // 16, zero_body, 0)
        tvec = thr_v[...]
        nvec = need_v[...]

        @plsc.parallel_loop(0, NREG, unroll=4,
                            carry=(jnp.int32(0), jnp.int32(0)))
        def body(i, carry):
            off, eq_off = carry
            u = buf[pl.ds(i * 16, 16)]
            v = u ^ ((u >> 31) & jnp.int32(0x7FFFFFFF))
            gt = v > tvec
            eq = v == tvec
            eqi = eq.astype(jnp.int32)
            pre = plsc.cumsum(eqi)
            eqo = jax.lax.broadcast(eq_off, (16,))
            sel_eq = eq & ((pre + eqo) <= nvec)
            m = gt | sel_eq
            mi = m.astype(jnp.int32)
            cm = plsc.cumsum(mi)
            idxv = jax.lax.broadcast(i * 16, (16,)) + lax.iota(jnp.int32, 16)
            pos = jnp.where(
                m,
                jax.lax.broadcast(off, (16,)) + cm - 1,
                jax.lax.broadcast(NPAD, (16,)) + lax.iota(jnp.int32, 16))
            plsc.store_scatter(obuf, [pos], idxv)
            return (off + jnp.sum(mi), eq_off + jnp.sum(eqi))
        pltpu.sync_copy(obuf.at[pl.ds(0, NPAD)],
                        idx_hbm.at[pl.ds(t * NPAD, NPAD)])


def _compact(scores_flat, thr_flat, need_flat):
    return pl.kernel(
        _compact_body,
        out_type=jax.ShapeDtypeStruct((NT * NPAD,), jnp.int32),
        mesh=_SC_MESH(),
        compiler_params=pltpu.CompilerParams(needs_layout_passes=False),
        scratch_types=[
            pltpu.VMEM((N,), jnp.int32),
            pltpu.VMEM((NPAD + 16,), jnp.int32),
            pltpu.VMEM((16,), jnp.int32),
            pltpu.VMEM((16,), jnp.int32),
        ],
    )(scores_flat, thr_flat, need_flat)


def _gather_body(feat_hbm, idx_hbm, out_hbm, idx_v, rows0, rows1, sem0, sem1):
    wid = lax.axis_index("s") * 2 + lax.axis_index("c")
    base = wid * GROWS
    pltpu.sync_copy(idx_hbm.at[pl.ds(base, GROWS)], idx_v)
    nch = GROWS // GCH
    bufs = (rows0, rows1)
    sems = (sem0, sem1)
    pend = [None, None]
    pend[0] = pltpu.async_copy(
        feat_hbm.at[idx_v.at[pl.ds(0, GCH)]], rows0, sem0)
    for j in range(nch):
        if j + 1 < nch:
            pend[(j + 1) % 2] = pltpu.async_copy(
                feat_hbm.at[idx_v.at[pl.ds((j + 1) * GCH, GCH)]],
                bufs[(j + 1) % 2], sems[(j + 1) % 2])
        pend[j % 2].wait()
        pltpu.sync_copy(bufs[j % 2], out_hbm.at[pl.ds(base + j * GCH, GCH)])


def _gather(feat, idxf):
    return pl.kernel(
        _gather_body,
        out_type=jax.ShapeDtypeStruct((NT * NPAD, D), jnp.float32),
        mesh=_SC_MESH(),
        compiler_params=pltpu.CompilerParams(needs_layout_passes=False),
        scratch_types=[
            pltpu.VMEM((GROWS,), jnp.int32),
            pltpu.VMEM((GCH, D), jnp.float32),
            pltpu.VMEM((GCH, D), jnp.float32),
            pltpu.SemaphoreType.DMA,
            pltpu.SemaphoreType.DMA,
        ],
    )(feat, idxf)


def _tok_body(g3_ref, pp_ref, Wq_ref, bq_ref, Wk_ref, bk_ref, Wv_ref, bv_ref,
              Wo_ref, bo_ref, g_ref, bn_ref, Wc_ref, bc_ref,
              tf_ref, tl_ref, lst, vst):
    c = pl.program_id(1)

    @pl.when(c < NCHUNK)
    def _():
        sel = g3_ref[0]
        q = pp_ref[0] @ Wq_ref[0] + bq_ref[0]
        kk_ = sel @ Wk_ref[0] + bk_ref[0]
        vv = sel @ Wv_ref[0] + bv_ref[0]
        l8 = jax.lax.dot_general(kk_ * q, _blockdiag(),
                                 (((1,), (1,)), ((), ()))) * (1.0 / 8.0)
        row = c * CHUNK + jax.lax.broadcasted_iota(jnp.int32, (CHUNK, H), 0)
        l8 = jnp.where(row < KK, l8, -1e30)
        lst[pl.ds(c * CHUNK, CHUNK), :] = l8
        vst[pl.ds(c * CHUNK, CHUNK), :] = vv

    @pl.when(c == NCHUNK)
    def _():
        l8 = lst[...]
        m = jnp.max(l8, axis=0, keepdims=True)
        e = jnp.exp(l8 - m)
        z = jnp.sum(e, axis=0, keepdims=True)
        p = e / z
        pexp = p @ _blockdiag()
        comb = jnp.sum(pexp * vst[...], axis=0, keepdims=True)
        o = comb @ Wo_ref[0] + bo_ref[0]
        of = _ln_rows(o, g_ref[0], bn_ref[0])
        tf_ref[0] = of
        tl_ref[0] = of @ Wc_ref[0] + bc_ref[0]


def _tok(g3, pp, tWq, tbq, tWk, tbk, tWv, tbv, tWo, tbo, tg, tbn, Wc, bc):
    per_tok_mat = pl.BlockSpec((1, D, D), lambda t, c: (t, 0, 0))
    per_tok_vec = pl.BlockSpec((1, 1, D), lambda t, c: (t, 0, 0))
    return pl.pallas_call(
        _tok_body,
        grid=(NT, NCHUNK + 1),
        in_specs=[
            pl.BlockSpec((1, CHUNK, D),
                         lambda t, c: (t, jnp.minimum(c, NCHUNK - 1), 0)),
            per_tok_vec,
            per_tok_mat, per_tok_vec,
            per_tok_mat, per_tok_vec,
            per_tok_mat, per_tok_vec,
            per_tok_mat, per_tok_vec,
            per_tok_vec, per_tok_vec,
            pl.BlockSpec((1, D, 128), lambda t, c: (t, 0, 0)),
            pl.BlockSpec((1, 1, 128), lambda t, c: (t, 0, 0)),
        ],
        out_specs=[
            pl.BlockSpec((1, 1, D), lambda t, c: (t, 0, 0)),
            pl.BlockSpec((1, 1, 128), lambda t, c: (t, 0, 0)),
        ],
        out_shape=[
            jax.ShapeDtypeStruct((NT, 1, D), jnp.float32),
            jax.ShapeDtypeStruct((NT, 1, 128), jnp.float32),
        ],
        scratch_shapes=[
            pltpu.VMEM((NPAD, H), jnp.float32),
            pltpu.VMEM((NPAD, D), jnp.float32),
        ],
    )(g3, pp, tWq, tbq, tWk, tbk, tWv, tbv, tWo, tbo, tg, tbn, Wc, bc)


def _agg_body(tf_ref, qsel_ref, fs_ref, Wk_ref, bk_ref, Wv_ref, bv_ref,
              Wo_ref, bo_ref, g_ref, bn_ref, Ws_ref, bs_ref, out_ref):
    E_ = _blockdiag()
    ksum = fs_ref[...] @ Wk_ref[...] + N * bk_ref[...]
    tmp = qsel_ref[...] * ksum
    msel = jax.lax.dot_general(tmp, E_, (((1,), (1,)), ((), ()))) * (
        1.0 / (HD ** 0.5) / N)
    mmax = jnp.max(msel, axis=0, keepdims=True)
    e = jnp.exp(msel - mmax)
    w = e / jnp.sum(e, axis=0, keepdims=True)
    vp = tf_ref[...] @ Wv_ref[...] + bv_ref[...]
    wexp = w @ E_
    comb = jnp.sum(wexp * vp, axis=0, keepdims=True)
    o = comb @ Wo_ref[...] + bo_ref[...]
    of = _ln_rows(o, g_ref[...], bn_ref[...])
    out_ref[...] = of @ Ws_ref[...] + bs_ref[...]


def _agg(tf, qsel, fs, Wk, bk, Wv, bv, Wo, bo, g, bn, Ws, bs):
    return pl.pallas_call(
        _agg_body,
        out_shape=jax.ShapeDtypeStruct((1, 128), jnp.float32),
    )(tf, qsel, fs, Wk, bk, Wv, bv, Wo, bo, g, bn, Ws, bs)


def kernel(x, W_fe, b_fe, g_norm, bn_norm, W_proj, b_proj, prototypes,
           sel_Wq, sel_bq, sel_Wk, sel_bk, sel_Wv, sel_bv, sel_Wo, sel_bo,
           sel_g, sel_bn,
           tok_Wq, tok_bq, tok_Wk, tok_bk, tok_Wv, tok_bv, tok_Wo, tok_bo,
           tok_g, tok_bn,
           agg_Wv, agg_bv, agg_Wo, agg_bo, agg_g, agg_bn,
           tok_cls_W, tok_cls_b, slide_W, slide_b):
    r2 = lambda a: a.reshape(1, -1)

    pp, qsel = _prep(prototypes, W_proj, r2(b_proj), sel_Wq, r2(sel_bq))
    feat, scores3, fs = _feat(x, W_fe, r2(b_fe), r2(g_norm), r2(bn_norm),
                              sel_Wk, r2(sel_bk), qsel)
    thr, need = _thr(scores3)
    scores_i = jax.lax.bitcast_convert_type(scores3, jnp.int32)
    scf = scores_i.transpose(1, 0, 2).reshape(NT * N)
    idx = _compact(scf, thr.reshape(-1), need.reshape(-1))
    g3 = _gather(feat, idx).reshape(NT, NPAD, D)

    Wc = jnp.pad(tok_cls_W, ((0, 0), (0, 0), (0, 126)))
    bc = jnp.pad(tok_cls_b, ((0, 0), (0, 126))).reshape(NT, 1, 128)
    r3 = lambda a: a.reshape(NT, 1, D)
    tf, tl = _tok(g3, pp.reshape(NT, 1, D), tok_Wq, r3(tok_bq), tok_Wk,
                  r3(tok_bk), tok_Wv, r3(tok_bv), tok_Wo, r3(tok_bo),
                  r3(tok_g), r3(tok_bn), Wc, bc)

    Ws = jnp.pad(slide_W, ((0, 0), (0, 126)))
    bs = jnp.pad(r2(slide_b), ((0, 0), (0, 126)))
    sl = _agg(tf.reshape(NT, D), qsel, fs, sel_Wk, r2(sel_bk), agg_Wv,
              r2(agg_bv), agg_Wo, r2(agg_bo), r2(agg_g), r2(agg_bn), Ws, bs)

    return sl[:, :2], tl.reshape(NT, 128)[:, :2].reshape(NT, 1, 2)
